# Initial kernel scaffold; baseline (speedup 1.0000x reference)
import jax
import jax.numpy as jnp
from jax import lax
from jax.experimental import pallas as pl
from jax.experimental.pallas import tpu as pltpu
from jax.experimental.pallas import tpu_sc as plsc

NC = 2
NS = 16
L = 16
NW = NC * NS


def _sc_partials(h_flat, pos_flat, batch_idx, n, d, b):
    chunk = ((n + NW - 1) // NW + L - 1) // L * L
    groups = chunk // L
    last_base = n - chunk
    assert last_base % 8 == 0 and chunk % 8 == 0

    mesh = plsc.VectorSubcoreMesh(
        core_axis_name="c", subcore_axis_name="s", num_cores=NC, num_subcores=NS
    )

    @pl.kernel(
        out_type=jax.ShapeDtypeStruct((NW, 4 * b), jnp.float32),
        mesh=mesh,
        scratch_types=[
            pltpu.VMEM((chunk * d,), jnp.float32),
            pltpu.VMEM((chunk * 3,), jnp.float32),
            pltpu.VMEM((chunk,), jnp.int32),
            pltpu.VMEM((4 * b,), jnp.float32),
        ],
    )
    def sc_kernel(h_hbm, pos_hbm, idx_hbm, out_hbm, h_v, pos_v, idx_v, acc_v):
        wid = lax.axis_index("s") * NC + lax.axis_index("c")
        start = wid * chunk
        base = jnp.minimum(start, last_base)
        delta = start - base

        pltpu.sync_copy(h_hbm.at[pl.ds(base * d, chunk * d)], h_v)
        pltpu.sync_copy(pos_hbm.at[pl.ds(base * 3, chunk * 3)], pos_v)
        pltpu.sync_copy(idx_hbm.at[pl.ds(base, chunk)], idx_v)

        zeros = jnp.zeros((L,), jnp.float32)
        for i in range(4 * b // L):
            acc_v[pl.ds(i * L, L)] = zeros

        iota = lax.iota(jnp.int32, L)
        ones = jnp.ones((L,), jnp.float32)

        def body(g, carry):
            rows = g * L + iota
            bidx = idx_v[pl.ds(g * L, L)]
            c0 = plsc.load_gather(h_v, [rows * d + 3])
            c1 = plsc.load_gather(h_v, [rows * d + 4])
            c2 = plsc.load_gather(h_v, [rows * d + 5])
            c3 = plsc.load_gather(h_v, [rows * d + 6])
            cond = (c1 > c0) & (c1 >= c2) & (c1 >= c3) & (rows >= delta)
            px = plsc.load_gather(pos_v, [rows * 3])
            py = plsc.load_gather(pos_v, [rows * 3 + 1])
            pz = plsc.load_gather(pos_v, [rows * 3 + 2])
            plsc.addupdate_scatter(acc_v, [bidx], ones, mask=cond)
            plsc.addupdate_scatter(acc_v, [bidx + b], px, mask=cond)
            plsc.addupdate_scatter(acc_v, [bidx + 2 * b], py, mask=cond)
            plsc.addupdate_scatter(acc_v, [bidx + 3 * b], pz, mask=cond)
            return carry

        lax.fori_loop(0, groups, body, 0)
        pltpu.sync_copy(acc_v, out_hbm.at[wid])

    return sc_kernel(h_flat, pos_flat, batch_idx)


def _tc_combine(partials, b):
    def body(p_ref, pt_ref, pd_ref):
        s = jnp.sum(p_ref[...], axis=0, keepdims=True)
        cnt = s[:, 0:b]
        sx = s[:, b:2 * b]
        sy = s[:, 2 * b:3 * b]
        sz = s[:, 3 * b:4 * b]
        c = jnp.maximum(cnt, 1.0)
        mx, my, mz = sx / c, sy / c, sz / c
        pt = jnp.sqrt(mx * mx + my * my + mz * mz)
        pt_ref[...] = pt
        pd_ref[...] = jnp.concatenate([mx / pt, my / pt, mz / pt], axis=0)

    return pl.pallas_call(
        body,
        out_shape=[
            jax.ShapeDtypeStruct((1, b), jnp.float32),
            jax.ShapeDtypeStruct((3, b), jnp.float32),
        ],
    )(partials)


def kernel(x_global_features, h, pos_pxpypz_at_vertex, batch_idx):
    n, d = h.shape
    b = x_global_features.shape[0]
    partials = _sc_partials(
        h.reshape(-1), pos_pxpypz_at_vertex.reshape(-1), batch_idx, n, d, b
    )
    pt, pd = _tc_combine(partials, b)
    return pt.reshape(b), pd.T


# speedup vs baseline: 3.2097x; 3.2097x over previous
import jax
import jax.numpy as jnp
from jax import lax
from jax.experimental import pallas as pl
from jax.experimental.pallas import tpu as pltpu
from jax.experimental.pallas import tpu_sc as plsc

NC = 2
NS = 16
L = 16
NW = NC * NS


def _sc_partials(h_flat, pos_flat, batch_idx, n, d, b):
    chunk = ((n + NW - 1) // NW + L - 1) // L * L
    groups = chunk // L
    last_base = n - chunk
    assert last_base % 8 == 0 and chunk % 8 == 0

    mesh = plsc.VectorSubcoreMesh(
        core_axis_name="c", subcore_axis_name="s", num_cores=NC, num_subcores=NS
    )

    @pl.kernel(
        out_type=jax.ShapeDtypeStruct((NW, 4 * b), jnp.float32),
        mesh=mesh,
        scratch_types=[
            pltpu.VMEM((chunk * d,), jnp.float32),
            pltpu.VMEM((chunk * 3,), jnp.float32),
            pltpu.VMEM((chunk,), jnp.int32),
            pltpu.VMEM((4 * b,), jnp.float32),
        ],
        compiler_params=pltpu.CompilerParams(needs_layout_passes=False),
    )
    def sc_kernel(h_hbm, pos_hbm, idx_hbm, out_hbm, h_v, pos_v, idx_v, acc_v):
        wid = lax.axis_index("s") * NC + lax.axis_index("c")
        start = wid * chunk
        base = jnp.minimum(start, last_base)
        delta = start - base

        pltpu.sync_copy(h_hbm.at[pl.ds(base * d, chunk * d)], h_v)
        pltpu.sync_copy(pos_hbm.at[pl.ds(base * 3, chunk * 3)], pos_v)
        pltpu.sync_copy(idx_hbm.at[pl.ds(base, chunk)], idx_v)

        zeros = jnp.zeros((L,), jnp.float32)
        for i in range(4 * b // L):
            acc_v[pl.ds(i * L, L)] = zeros

        iota = lax.iota(jnp.int32, L)
        ones = jnp.ones((L,), jnp.float32)

        def body(g, carry):
            rows = g * L + iota
            bidx = idx_v[pl.ds(g * L, L)]
            c0 = plsc.load_gather(h_v, [rows * d + 3])
            c1 = plsc.load_gather(h_v, [rows * d + 4])
            c2 = plsc.load_gather(h_v, [rows * d + 5])
            c3 = plsc.load_gather(h_v, [rows * d + 6])
            cond = (c1 > c0) & (c1 >= c2) & (c1 >= c3) & (rows >= delta)
            px = plsc.load_gather(pos_v, [rows * 3])
            py = plsc.load_gather(pos_v, [rows * 3 + 1])
            pz = plsc.load_gather(pos_v, [rows * 3 + 2])
            plsc.addupdate_scatter(acc_v, [bidx], ones, mask=cond)
            plsc.addupdate_scatter(acc_v, [bidx + b], px, mask=cond)
            plsc.addupdate_scatter(acc_v, [bidx + 2 * b], py, mask=cond)
            plsc.addupdate_scatter(acc_v, [bidx + 3 * b], pz, mask=cond)
            return carry

        lax.fori_loop(0, groups, body, 0)
        pltpu.sync_copy(acc_v, out_hbm.at[wid])

    return sc_kernel(h_flat, pos_flat, batch_idx)


def _tc_combine(partials, b):
    def body(p_ref, pt_ref, pd_ref):
        s = jnp.sum(p_ref[...], axis=0, keepdims=True)
        cnt = s[:, 0:b]
        sx = s[:, b:2 * b]
        sy = s[:, 2 * b:3 * b]
        sz = s[:, 3 * b:4 * b]
        c = jnp.maximum(cnt, 1.0)
        mx, my, mz = sx / c, sy / c, sz / c
        pt = jnp.sqrt(mx * mx + my * my + mz * mz)
        pt_ref[...] = pt
        pd_ref[...] = jnp.concatenate([mx / pt, my / pt, mz / pt], axis=0)

    return pl.pallas_call(
        body,
        out_shape=[
            jax.ShapeDtypeStruct((1, b), jnp.float32),
            jax.ShapeDtypeStruct((3, b), jnp.float32),
        ],
    )(partials)


def kernel(x_global_features, h, pos_pxpypz_at_vertex, batch_idx):
    n, d = h.shape
    b = x_global_features.shape[0]
    partials = _sc_partials(
        h.reshape(-1), pos_pxpypz_at_vertex.reshape(-1), batch_idx, n, d, b
    )
    pt, pd = _tc_combine(partials, b)
    return pt.reshape(b), pd.T
